# baseline (speedup 1.0000x reference)
import functools

import jax
import jax.numpy as jnp
from jax import lax
from jax.experimental import pallas as pl
from jax.experimental.pallas import tpu as pltpu
from jax.experimental.pallas import tpu_sc as plsc

NC = 2
NS = 16
NW = NC * NS
LANES = 128
DEGW = 16
ROWBLK = 512


def _sc_degree(n_acc, k, f):
    rpt = n_acc // NS
    mesh = plsc.VectorSubcoreMesh(core_axis_name="c", subcore_axis_name="s",
                                  num_cores=NC, num_subcores=NS)

    @functools.partial(
        pl.kernel,
        out_type=jax.ShapeDtypeStruct((NC, n_acc, f), jnp.float32),
        mesh=mesh,
        scratch_types=[
            pltpu.VMEM((1, LANES), jnp.int32),
            pltpu.VMEM((1, LANES), jnp.int32),
            pltpu.VMEM((LANES, f), jnp.float32),
            pltpu.VMEM_SHARED((n_acc, f), jnp.float32),
            pltpu.SemaphoreType.DMA,
            pltpu.SemaphoreType.DMA,
        ],
    )
    def deg_kernel(dst_hbm, ones_hbm, zeros_hbm, out_hbm,
                   db0, db1, ones_v, acc_sh, ds0, ds1):
        c = lax.axis_index("c")
        s = lax.axis_index("s")
        wid = c * NS + s
        dbuf = (db0, db1)
        dsem = (ds0, ds1)
        pltpu.sync_copy(zeros_hbm.at[pl.ds(s * rpt, rpt)],
                        acc_sh.at[pl.ds(s * rpt, rpt)])
        pltpu.sync_copy(ones_hbm, ones_v)
        for b in range(2):
            pltpu.async_copy(dst_hbm.at[wid, b], dbuf[b], dsem[b])
        plsc.subcore_barrier()

        @pl.loop(0, k, step=2)
        def _(j):
            for b in range(2):
                jj = j + b
                pltpu.make_async_copy(
                    dst_hbm.at[wid, jj], dbuf[b], dsem[b]).wait()
                pltpu.sync_copy(ones_v, acc_sh.at[dbuf[b].at[0]], add=True)

                @pl.when(jj + 2 < k)
                def _():
                    pltpu.async_copy(dst_hbm.at[wid, jj + 2], dbuf[b], dsem[b])

        plsc.subcore_barrier()
        pltpu.sync_copy(acc_sh.at[pl.ds(s * rpt, rpt)],
                        out_hbm.at[c, pl.ds(s * rpt, rpt)])

    return deg_kernel


def _sc_propagate(n_acc, k, per_tile, f):
    rpt = n_acc // NS
    mesh = plsc.VectorSubcoreMesh(core_axis_name="c", subcore_axis_name="s",
                                  num_cores=NC, num_subcores=NS)

    @functools.partial(
        pl.kernel,
        out_type=jax.ShapeDtypeStruct((NC, n_acc, f), jnp.float32),
        mesh=mesh,
        scratch_types=[
            pltpu.VMEM((per_tile,), jnp.int32),
            pltpu.VMEM((1, LANES), jnp.int32),
            pltpu.VMEM((1, LANES), jnp.int32),
            pltpu.VMEM((LANES, f), jnp.float32),
            pltpu.VMEM((LANES, f), jnp.float32),
            pltpu.VMEM_SHARED((n_acc, f), jnp.float32),
            pltpu.SemaphoreType.DMA,
            pltpu.SemaphoreType.DMA,
            pltpu.SemaphoreType.DMA,
            pltpu.SemaphoreType.DMA,
        ],
    )
    def prop_kernel(src_hbm, dst_hbm, y_hbm, zeros_hbm, out_hbm,
                    src_v, db0, db1, gb0, gb1, acc_sh, gs0, gs1, ds0, ds1):
        c = lax.axis_index("c")
        s = lax.axis_index("s")
        wid = c * NS + s
        dbuf = (db0, db1)
        gbuf = (gb0, gb1)
        gsem = (gs0, gs1)
        dsem = (ds0, ds1)
        pltpu.sync_copy(zeros_hbm.at[pl.ds(s * rpt, rpt)],
                        acc_sh.at[pl.ds(s * rpt, rpt)])
        pltpu.sync_copy(src_hbm.at[pl.ds(wid * per_tile, per_tile)], src_v)
        for b in range(2):
            pltpu.async_copy(dst_hbm.at[wid, b], dbuf[b], dsem[b])
            pltpu.async_copy(y_hbm.at[src_v.at[pl.ds(b * LANES, LANES)]],
                             gbuf[b], gsem[b])
        plsc.subcore_barrier()

        @pl.loop(0, k, step=2)
        def _(j):
            for b in range(2):
                jj = j + b
                pltpu.make_async_copy(
                    y_hbm.at[src_v.at[pl.ds(jj * LANES, LANES)]],
                    gbuf[b], gsem[b]).wait()
                pltpu.make_async_copy(
                    dst_hbm.at[wid, jj], dbuf[b], dsem[b]).wait()
                pltpu.sync_copy(gbuf[b], acc_sh.at[dbuf[b].at[0]], add=True)

                @pl.when(jj + 2 < k)
                def _():
                    pltpu.async_copy(dst_hbm.at[wid, jj + 2], dbuf[b], dsem[b])
                    pltpu.async_copy(
                        y_hbm.at[src_v.at[pl.ds((jj + 2) * LANES, LANES)]],
                        gbuf[b], gsem[b])

        plsc.subcore_barrier()
        pltpu.sync_copy(acc_sh.at[pl.ds(s * rpt, rpt)],
                        out_hbm.at[c, pl.ds(s * rpt, rpt)])

    return prop_kernel


def _tc_stage1(n, f, grid):
    def body(x_ref, w_ref, d0_ref, d1_ref, y_ref, dinv_ref):
        deg = d0_ref[:, :1] + d1_ref[:, :1] + 1.0
        dinv = lax.rsqrt(deg)
        xw = jnp.dot(x_ref[...], w_ref[...], preferred_element_type=jnp.float32)
        y_ref[...] = xw * dinv
        dinv_ref[...] = dinv

    return pl.pallas_call(
        body,
        grid=(grid,),
        in_specs=[
            pl.BlockSpec((ROWBLK, f), lambda i: (i, 0)),
            pl.BlockSpec((f, f), lambda i: (0, 0)),
            pl.BlockSpec((ROWBLK, f), lambda i: (i, 0)),
            pl.BlockSpec((ROWBLK, f), lambda i: (i, 0)),
        ],
        out_specs=[
            pl.BlockSpec((ROWBLK, f), lambda i: (i, 0)),
            pl.BlockSpec((ROWBLK, 1), lambda i: (i, 0)),
        ],
        out_shape=[
            jax.ShapeDtypeStruct((n, f), jnp.float32),
            jax.ShapeDtypeStruct((n, 1), jnp.float32),
        ],
    )


def _tc_stage2(n, f, grid):
    def body(a0_ref, a1_ref, y_ref, dinv_ref, b_ref, w_ref, out_ref):
        dinv = dinv_ref[...]
        p = (a0_ref[...] + a1_ref[...] + y_ref[...]) * dinv + b_ref[...]
        h = jnp.maximum(p, 0.0)
        hw = jnp.dot(h, w_ref[...], preferred_element_type=jnp.float32)
        out_ref[...] = hw * dinv

    return pl.pallas_call(
        body,
        grid=(grid,),
        in_specs=[
            pl.BlockSpec((ROWBLK, f), lambda i: (i, 0)),
            pl.BlockSpec((ROWBLK, f), lambda i: (i, 0)),
            pl.BlockSpec((ROWBLK, f), lambda i: (i, 0)),
            pl.BlockSpec((ROWBLK, 1), lambda i: (i, 0)),
            pl.BlockSpec((1, f), lambda i: (0, 0)),
            pl.BlockSpec((f, f), lambda i: (0, 0)),
        ],
        out_specs=pl.BlockSpec((ROWBLK, f), lambda i: (i, 0)),
        out_shape=jax.ShapeDtypeStruct((n, f), jnp.float32),
    )


def _tc_stage3(n, f, c_out, grid):
    def body(a0_ref, a1_ref, y_ref, dinv_ref, b2_ref, wl_ref, bl_ref, out_ref):
        dinv = dinv_ref[...]
        p = (a0_ref[...] + a1_ref[...] + y_ref[...]) * dinv + b2_ref[...]
        h = jnp.maximum(p, 0.0)
        out_ref[...] = (
            jnp.dot(h, wl_ref[...], preferred_element_type=jnp.float32)
            + bl_ref[...]
        )

    return pl.pallas_call(
        body,
        grid=(grid,),
        in_specs=[
            pl.BlockSpec((ROWBLK, f), lambda i: (i, 0)),
            pl.BlockSpec((ROWBLK, f), lambda i: (i, 0)),
            pl.BlockSpec((ROWBLK, f), lambda i: (i, 0)),
            pl.BlockSpec((ROWBLK, 1), lambda i: (i, 0)),
            pl.BlockSpec((1, f), lambda i: (0, 0)),
            pl.BlockSpec((f, c_out), lambda i: (0, 0)),
            pl.BlockSpec((1, c_out), lambda i: (0, 0)),
        ],
        out_specs=pl.BlockSpec((ROWBLK, c_out), lambda i: (i, 0)),
        out_shape=jax.ShapeDtypeStruct((n, c_out), jnp.float32),
    )


def kernel(node_features, edge_indices, W1, b1, W2, b2, Wl, bl):
    n, f = node_features.shape
    e = edge_indices.shape[1]
    c_out = Wl.shape[0]

    per_tile = -(-e // NW)
    k = -(-per_tile // LANES)
    if k % 2:
        k += 1
    per_tile = k * LANES
    pad = NW * per_tile - e
    n_acc = ((n + 1 + 255) // 256) * 256

    trash = n + (jnp.arange(pad, dtype=jnp.int32) % (n_acc - n))
    src_flat = jnp.concatenate([edge_indices[0], jnp.zeros((pad,), jnp.int32)])
    dst_flat = jnp.concatenate([edge_indices[1], trash])
    dst4 = dst_flat.reshape(NW, k, 1, LANES)

    zeros_acc = jnp.zeros((n_acc, f), jnp.float32)
    ones_rows = jnp.ones((LANES, f), jnp.float32)

    grid = -(-n // ROWBLK)

    degp = _sc_degree(n_acc, k, f)(dst4, ones_rows, zeros_acc)

    W1t = jnp.transpose(W1)
    W2t = jnp.transpose(W2)
    Wlt = jnp.transpose(Wl)

    y1, dinv = _tc_stage1(n, f, grid)(
        node_features, W1t, degp[0, :n], degp[1, :n])

    prop = _sc_propagate(n_acc, k, per_tile, f)
    acc1 = prop(src_flat, dst4, y1, zeros_acc)
    y2 = _tc_stage2(n, f, grid)(
        acc1[0, :n], acc1[1, :n], y1, dinv, b1.reshape(1, f), W2t)

    acc2 = prop(src_flat, dst4, y2, zeros_acc)
    out = _tc_stage3(n, f, c_out, grid)(
        acc2[0, :n], acc2[1, :n], y2, dinv, b2.reshape(1, f), Wlt,
        bl.reshape(1, c_out))
    return out


# speedup vs baseline: 1.1034x; 1.1034x over previous
import functools

import jax
import jax.numpy as jnp
from jax import lax
from jax.experimental import pallas as pl
from jax.experimental.pallas import tpu as pltpu
from jax.experimental.pallas import tpu_sc as plsc

NC = 2
NS = 16
NW = NC * NS
LANES = 128
PCH = 64
PDEPTH = 4
DEGW = 16
ROWBLK = 512


def _sc_degree(n_acc, k, f):
    rpt = n_acc // NS
    mesh = plsc.VectorSubcoreMesh(core_axis_name="c", subcore_axis_name="s",
                                  num_cores=NC, num_subcores=NS)

    @functools.partial(
        pl.kernel,
        out_type=jax.ShapeDtypeStruct((NC, n_acc, f), jnp.float32),
        mesh=mesh,
        scratch_types=[
            pltpu.VMEM((1, LANES), jnp.int32),
            pltpu.VMEM((1, LANES), jnp.int32),
            pltpu.VMEM((LANES, f), jnp.float32),
            pltpu.VMEM_SHARED((n_acc, f), jnp.float32),
            pltpu.SemaphoreType.DMA,
            pltpu.SemaphoreType.DMA,
        ],
    )
    def deg_kernel(dst_hbm, ones_hbm, zeros_hbm, out_hbm,
                   db0, db1, ones_v, acc_sh, ds0, ds1):
        c = lax.axis_index("c")
        s = lax.axis_index("s")
        wid = c * NS + s
        dbuf = (db0, db1)
        dsem = (ds0, ds1)
        pltpu.sync_copy(zeros_hbm.at[pl.ds(s * rpt, rpt)],
                        acc_sh.at[pl.ds(s * rpt, rpt)])
        pltpu.sync_copy(ones_hbm, ones_v)
        for b in range(2):
            pltpu.async_copy(dst_hbm.at[wid, b], dbuf[b], dsem[b])
        plsc.subcore_barrier()

        @pl.loop(0, k, step=2)
        def _(j):
            for b in range(2):
                jj = j + b
                pltpu.make_async_copy(
                    dst_hbm.at[wid, jj], dbuf[b], dsem[b]).wait()
                pltpu.sync_copy(ones_v, acc_sh.at[dbuf[b].at[0]], add=True)

                @pl.when(jj + 2 < k)
                def _():
                    pltpu.async_copy(dst_hbm.at[wid, jj + 2], dbuf[b], dsem[b])

        plsc.subcore_barrier()
        pltpu.sync_copy(acc_sh.at[pl.ds(s * rpt, rpt)],
                        out_hbm.at[c, pl.ds(s * rpt, rpt)])

    return deg_kernel


def _sc_propagate(n_acc, per_tile, f):
    rpt = n_acc // NS
    kc = per_tile // PCH
    mesh = plsc.VectorSubcoreMesh(core_axis_name="c", subcore_axis_name="s",
                                  num_cores=NC, num_subcores=NS)

    @functools.partial(
        pl.kernel,
        out_type=jax.ShapeDtypeStruct((NC, n_acc, f), jnp.float32),
        mesh=mesh,
        scratch_types=[
            pltpu.VMEM((per_tile,), jnp.int32),
            [pltpu.VMEM((1, PCH), jnp.int32) for _ in range(PDEPTH)],
            [pltpu.VMEM((PCH, f), jnp.float32) for _ in range(PDEPTH)],
            pltpu.VMEM_SHARED((n_acc, f), jnp.float32),
            [pltpu.SemaphoreType.DMA for _ in range(PDEPTH)],
            [pltpu.SemaphoreType.DMA for _ in range(PDEPTH)],
        ],
    )
    def prop_kernel(src_hbm, dst_hbm, y_hbm, zeros_hbm, out_hbm,
                    src_v, dbuf, gbuf, acc_sh, dsem, gsem):
        c = lax.axis_index("c")
        s = lax.axis_index("s")
        wid = c * NS + s
        pltpu.sync_copy(zeros_hbm.at[pl.ds(s * rpt, rpt)],
                        acc_sh.at[pl.ds(s * rpt, rpt)])
        pltpu.sync_copy(src_hbm.at[pl.ds(wid * per_tile, per_tile)], src_v)
        for b in range(PDEPTH):
            pltpu.async_copy(dst_hbm.at[wid, b], dbuf[b], dsem[b])
            pltpu.async_copy(y_hbm.at[src_v.at[pl.ds(b * PCH, PCH)]],
                             gbuf[b], gsem[b])
        plsc.subcore_barrier()

        @pl.loop(0, kc, step=PDEPTH)
        def _(j):
            for b in range(PDEPTH):
                jj = j + b
                pltpu.make_async_copy(
                    y_hbm.at[src_v.at[pl.ds(jj * PCH, PCH)]],
                    gbuf[b], gsem[b]).wait()
                pltpu.make_async_copy(
                    dst_hbm.at[wid, jj], dbuf[b], dsem[b]).wait()
                pltpu.sync_copy(gbuf[b], acc_sh.at[dbuf[b].at[0]], add=True)

                @pl.when(jj + PDEPTH < kc)
                def _():
                    pltpu.async_copy(dst_hbm.at[wid, jj + PDEPTH],
                                     dbuf[b], dsem[b])
                    pltpu.async_copy(
                        y_hbm.at[src_v.at[pl.ds((jj + PDEPTH) * PCH, PCH)]],
                        gbuf[b], gsem[b])

        plsc.subcore_barrier()
        pltpu.sync_copy(acc_sh.at[pl.ds(s * rpt, rpt)],
                        out_hbm.at[c, pl.ds(s * rpt, rpt)])

    return prop_kernel


def _tc_stage1(n, f, grid):
    def body(x_ref, w_ref, d0_ref, d1_ref, y_ref, dinv_ref):
        deg = d0_ref[:, :1] + d1_ref[:, :1] + 1.0
        dinv = lax.rsqrt(deg)
        xw = jnp.dot(x_ref[...], w_ref[...], preferred_element_type=jnp.float32)
        y_ref[...] = xw * dinv
        dinv_ref[...] = dinv

    return pl.pallas_call(
        body,
        grid=(grid,),
        in_specs=[
            pl.BlockSpec((ROWBLK, f), lambda i: (i, 0)),
            pl.BlockSpec((f, f), lambda i: (0, 0)),
            pl.BlockSpec((ROWBLK, f), lambda i: (i, 0)),
            pl.BlockSpec((ROWBLK, f), lambda i: (i, 0)),
        ],
        out_specs=[
            pl.BlockSpec((ROWBLK, f), lambda i: (i, 0)),
            pl.BlockSpec((ROWBLK, 1), lambda i: (i, 0)),
        ],
        out_shape=[
            jax.ShapeDtypeStruct((n, f), jnp.float32),
            jax.ShapeDtypeStruct((n, 1), jnp.float32),
        ],
    )


def _tc_stage2(n, f, grid):
    def body(a0_ref, a1_ref, y_ref, dinv_ref, b_ref, w_ref, out_ref):
        dinv = dinv_ref[...]
        p = (a0_ref[...] + a1_ref[...] + y_ref[...]) * dinv + b_ref[...]
        h = jnp.maximum(p, 0.0)
        hw = jnp.dot(h, w_ref[...], preferred_element_type=jnp.float32)
        out_ref[...] = hw * dinv

    return pl.pallas_call(
        body,
        grid=(grid,),
        in_specs=[
            pl.BlockSpec((ROWBLK, f), lambda i: (i, 0)),
            pl.BlockSpec((ROWBLK, f), lambda i: (i, 0)),
            pl.BlockSpec((ROWBLK, f), lambda i: (i, 0)),
            pl.BlockSpec((ROWBLK, 1), lambda i: (i, 0)),
            pl.BlockSpec((1, f), lambda i: (0, 0)),
            pl.BlockSpec((f, f), lambda i: (0, 0)),
        ],
        out_specs=pl.BlockSpec((ROWBLK, f), lambda i: (i, 0)),
        out_shape=jax.ShapeDtypeStruct((n, f), jnp.float32),
    )


def _tc_stage3(n, f, c_out, grid):
    def body(a0_ref, a1_ref, y_ref, dinv_ref, b2_ref, wl_ref, bl_ref, out_ref):
        dinv = dinv_ref[...]
        p = (a0_ref[...] + a1_ref[...] + y_ref[...]) * dinv + b2_ref[...]
        h = jnp.maximum(p, 0.0)
        out_ref[...] = (
            jnp.dot(h, wl_ref[...], preferred_element_type=jnp.float32)
            + bl_ref[...]
        )

    return pl.pallas_call(
        body,
        grid=(grid,),
        in_specs=[
            pl.BlockSpec((ROWBLK, f), lambda i: (i, 0)),
            pl.BlockSpec((ROWBLK, f), lambda i: (i, 0)),
            pl.BlockSpec((ROWBLK, f), lambda i: (i, 0)),
            pl.BlockSpec((ROWBLK, 1), lambda i: (i, 0)),
            pl.BlockSpec((1, f), lambda i: (0, 0)),
            pl.BlockSpec((f, c_out), lambda i: (0, 0)),
            pl.BlockSpec((1, c_out), lambda i: (0, 0)),
        ],
        out_specs=pl.BlockSpec((ROWBLK, c_out), lambda i: (i, 0)),
        out_shape=jax.ShapeDtypeStruct((n, c_out), jnp.float32),
    )


def kernel(node_features, edge_indices, W1, b1, W2, b2, Wl, bl):
    n, f = node_features.shape
    e = edge_indices.shape[1]
    c_out = Wl.shape[0]

    per_tile = -(-e // NW)
    k = -(-per_tile // LANES)
    if k % 2:
        k += 1
    per_tile = k * LANES
    pad = NW * per_tile - e
    n_acc = ((n + 1 + 255) // 256) * 256

    trash = n + (jnp.arange(pad, dtype=jnp.int32) % (n_acc - n))
    src_flat = jnp.concatenate([edge_indices[0], jnp.zeros((pad,), jnp.int32)])
    dst_flat = jnp.concatenate([edge_indices[1], trash])
    dst4 = dst_flat.reshape(NW, k, 1, LANES)
    dst4p = dst_flat.reshape(NW, per_tile // PCH, 1, PCH)

    zeros_acc = jnp.zeros((n_acc, f), jnp.float32)
    ones_rows = jnp.ones((LANES, f), jnp.float32)

    grid = -(-n // ROWBLK)

    degp = _sc_degree(n_acc, k, f)(dst4, ones_rows, zeros_acc)

    W1t = jnp.transpose(W1)
    W2t = jnp.transpose(W2)
    Wlt = jnp.transpose(Wl)

    y1, dinv = _tc_stage1(n, f, grid)(
        node_features, W1t, degp[0, :n], degp[1, :n])

    prop = _sc_propagate(n_acc, per_tile, f)
    acc1 = prop(src_flat, dst4p, y1, zeros_acc)
    y2 = _tc_stage2(n, f, grid)(
        acc1[0, :n], acc1[1, :n], y1, dinv, b1.reshape(1, f), W2t)

    acc2 = prop(src_flat, dst4p, y2, zeros_acc)
    out = _tc_stage3(n, f, c_out, grid)(
        acc2[0, :n], acc2[1, :n], y2, dinv, b2.reshape(1, f), Wlt,
        bl.reshape(1, c_out))
    return out


# speedup vs baseline: 2.1030x; 1.9060x over previous
import functools

import jax
import jax.numpy as jnp
from jax import lax
from jax.experimental import pallas as pl
from jax.experimental.pallas import tpu as pltpu
from jax.experimental.pallas import tpu_sc as plsc

NC = 2
NS = 16
NW = NC * NS
LANES = 128
PCH = 64
PDEPTH = 3
FRAC0 = 0.81
DEGW = 16
ROWBLK = 512


def _sc_degree(n_acc, k, f):
    rpt = n_acc // NS
    mesh = plsc.VectorSubcoreMesh(core_axis_name="c", subcore_axis_name="s",
                                  num_cores=NC, num_subcores=NS)

    @functools.partial(
        pl.kernel,
        out_type=jax.ShapeDtypeStruct((NC, n_acc, f), jnp.float32),
        mesh=mesh,
        scratch_types=[
            pltpu.VMEM((1, LANES), jnp.int32),
            pltpu.VMEM((1, LANES), jnp.int32),
            pltpu.VMEM((LANES, f), jnp.float32),
            pltpu.VMEM_SHARED((n_acc, f), jnp.float32),
            pltpu.SemaphoreType.DMA,
            pltpu.SemaphoreType.DMA,
        ],
    )
    def deg_kernel(dst_hbm, ones_hbm, zeros_hbm, out_hbm,
                   db0, db1, ones_v, acc_sh, ds0, ds1):
        c = lax.axis_index("c")
        s = lax.axis_index("s")
        wid = c * NS + s
        dbuf = (db0, db1)
        dsem = (ds0, ds1)
        pltpu.sync_copy(zeros_hbm.at[pl.ds(s * rpt, rpt)],
                        acc_sh.at[pl.ds(s * rpt, rpt)])
        pltpu.sync_copy(ones_hbm, ones_v)
        for b in range(2):
            pltpu.async_copy(dst_hbm.at[wid, b], dbuf[b], dsem[b])
        plsc.subcore_barrier()

        @pl.loop(0, k, step=2)
        def _(j):
            for b in range(2):
                jj = j + b
                pltpu.make_async_copy(
                    dst_hbm.at[wid, jj], dbuf[b], dsem[b]).wait()
                pltpu.sync_copy(ones_v, acc_sh.at[dbuf[b].at[0]], add=True)

                @pl.when(jj + 2 < k)
                def _():
                    pltpu.async_copy(dst_hbm.at[wid, jj + 2], dbuf[b], dsem[b])

        plsc.subcore_barrier()
        pltpu.sync_copy(acc_sh.at[pl.ds(s * rpt, rpt)],
                        out_hbm.at[c, pl.ds(s * rpt, rpt)])

    return deg_kernel


def _sc_propagate(n_acc, pt0, pt1, f):
    rpt = n_acc // NS
    kc0 = pt0 // PCH
    kc1 = pt1 // PCH
    mesh = plsc.VectorSubcoreMesh(core_axis_name="c", subcore_axis_name="s",
                                  num_cores=NC, num_subcores=NS)

    @functools.partial(
        pl.kernel,
        out_type=jax.ShapeDtypeStruct((NC, n_acc, f), jnp.float32),
        mesh=mesh,
        scratch_types=[
            pltpu.VMEM((pt0,), jnp.int32),
            [pltpu.VMEM((1, PCH), jnp.int32) for _ in range(PDEPTH)],
            [pltpu.VMEM((PCH, f), jnp.float32) for _ in range(PDEPTH)],
            pltpu.VMEM_SHARED((n_acc, f), jnp.float32),
            [pltpu.SemaphoreType.DMA for _ in range(PDEPTH)],
            [pltpu.SemaphoreType.DMA for _ in range(PDEPTH)],
        ],
    )
    def prop_kernel(src_hbm, dst_hbm, y_hbm, zeros_hbm, out_hbm,
                    src_v, dbuf, gbuf, acc_sh, dsem, gsem):
        c = lax.axis_index("c")
        s = lax.axis_index("s")
        wid = c * NS + s
        kc = jnp.where(c == 0, kc0, kc1)
        base = jnp.where(c == 0, s * pt0, NS * pt0 + s * pt1)
        pltpu.sync_copy(zeros_hbm.at[pl.ds(s * rpt, rpt)],
                        acc_sh.at[pl.ds(s * rpt, rpt)])
        pltpu.sync_copy(src_hbm.at[pl.ds(base, pt0)], src_v)
        for b in range(PDEPTH):
            pltpu.async_copy(dst_hbm.at[wid, b], dbuf[b], dsem[b])
            pltpu.async_copy(y_hbm.at[src_v.at[pl.ds(b * PCH, PCH)]],
                             gbuf[b], gsem[b])
        plsc.subcore_barrier()

        @pl.loop(0, kc, step=PDEPTH)
        def _(j):
            for b in range(PDEPTH):
                jj = j + b
                pltpu.make_async_copy(
                    y_hbm.at[src_v.at[pl.ds(jj * PCH, PCH)]],
                    gbuf[b], gsem[b]).wait()
                pltpu.make_async_copy(
                    dst_hbm.at[wid, jj], dbuf[b], dsem[b]).wait()
                pltpu.sync_copy(gbuf[b], acc_sh.at[dbuf[b].at[0]], add=True)

                @pl.when(jj + PDEPTH < kc)
                def _():
                    pltpu.async_copy(dst_hbm.at[wid, jj + PDEPTH],
                                     dbuf[b], dsem[b])
                    pltpu.async_copy(
                        y_hbm.at[src_v.at[pl.ds((jj + PDEPTH) * PCH, PCH)]],
                        gbuf[b], gsem[b])

        plsc.subcore_barrier()
        pltpu.sync_copy(acc_sh.at[pl.ds(s * rpt, rpt)],
                        out_hbm.at[c, pl.ds(s * rpt, rpt)])

    return prop_kernel


def _tc_stage1(n, f, grid):
    def body(x_ref, w_ref, d0_ref, d1_ref, y_ref, dinv_ref):
        deg = d0_ref[:, :1] + d1_ref[:, :1] + 1.0
        dinv = lax.rsqrt(deg)
        xw = jnp.dot(x_ref[...], w_ref[...], preferred_element_type=jnp.float32)
        y_ref[...] = xw * dinv
        dinv_ref[...] = dinv

    return pl.pallas_call(
        body,
        grid=(grid,),
        in_specs=[
            pl.BlockSpec((ROWBLK, f), lambda i: (i, 0)),
            pl.BlockSpec((f, f), lambda i: (0, 0)),
            pl.BlockSpec((ROWBLK, f), lambda i: (i, 0)),
            pl.BlockSpec((ROWBLK, f), lambda i: (i, 0)),
        ],
        out_specs=[
            pl.BlockSpec((ROWBLK, f), lambda i: (i, 0)),
            pl.BlockSpec((ROWBLK, 1), lambda i: (i, 0)),
        ],
        out_shape=[
            jax.ShapeDtypeStruct((n, f), jnp.float32),
            jax.ShapeDtypeStruct((n, 1), jnp.float32),
        ],
    )


def _tc_stage2(n, f, grid):
    def body(a0_ref, a1_ref, y_ref, dinv_ref, b_ref, w_ref, out_ref):
        dinv = dinv_ref[...]
        p = (a0_ref[...] + a1_ref[...] + y_ref[...]) * dinv + b_ref[...]
        h = jnp.maximum(p, 0.0)
        hw = jnp.dot(h, w_ref[...], preferred_element_type=jnp.float32)
        out_ref[...] = hw * dinv

    return pl.pallas_call(
        body,
        grid=(grid,),
        in_specs=[
            pl.BlockSpec((ROWBLK, f), lambda i: (i, 0)),
            pl.BlockSpec((ROWBLK, f), lambda i: (i, 0)),
            pl.BlockSpec((ROWBLK, f), lambda i: (i, 0)),
            pl.BlockSpec((ROWBLK, 1), lambda i: (i, 0)),
            pl.BlockSpec((1, f), lambda i: (0, 0)),
            pl.BlockSpec((f, f), lambda i: (0, 0)),
        ],
        out_specs=pl.BlockSpec((ROWBLK, f), lambda i: (i, 0)),
        out_shape=jax.ShapeDtypeStruct((n, f), jnp.float32),
    )


def _tc_stage3(n, f, c_out, grid):
    def body(a0_ref, a1_ref, y_ref, dinv_ref, b2_ref, wl_ref, bl_ref, out_ref):
        dinv = dinv_ref[...]
        p = (a0_ref[...] + a1_ref[...] + y_ref[...]) * dinv + b2_ref[...]
        h = jnp.maximum(p, 0.0)
        out_ref[...] = (
            jnp.dot(h, wl_ref[...], preferred_element_type=jnp.float32)
            + bl_ref[...]
        )

    return pl.pallas_call(
        body,
        grid=(grid,),
        in_specs=[
            pl.BlockSpec((ROWBLK, f), lambda i: (i, 0)),
            pl.BlockSpec((ROWBLK, f), lambda i: (i, 0)),
            pl.BlockSpec((ROWBLK, f), lambda i: (i, 0)),
            pl.BlockSpec((ROWBLK, 1), lambda i: (i, 0)),
            pl.BlockSpec((1, f), lambda i: (0, 0)),
            pl.BlockSpec((f, c_out), lambda i: (0, 0)),
            pl.BlockSpec((1, c_out), lambda i: (0, 0)),
        ],
        out_specs=pl.BlockSpec((ROWBLK, c_out), lambda i: (i, 0)),
        out_shape=jax.ShapeDtypeStruct((n, c_out), jnp.float32),
    )


def kernel(node_features, edge_indices, W1, b1, W2, b2, Wl, bl):
    n, f = node_features.shape
    e = edge_indices.shape[1]
    c_out = Wl.shape[0]

    n_acc = ((n + 1 + 255) // 256) * 256

    unit = PCH * PDEPTH
    pt0 = max(unit, int(round(FRAC0 * e / NS / unit)) * unit)
    pt1 = max(unit, -(-(e - NS * pt0) // (NS * unit)) * unit)
    kc0, kc1 = pt0 // PCH, pt1 // PCH
    l0, l1 = NS * pt0, NS * pt1
    pad = l0 + l1 - e

    def padded(arr, m, fill_trash):
        if fill_trash:
            fill = n + (jnp.arange(m, dtype=jnp.int32) % (n_acc - n))
        else:
            fill = jnp.zeros((m,), jnp.int32)
        return jnp.concatenate([arr, fill])

    src_flat = padded(edge_indices[0], pad, False)
    dst_flat = padded(edge_indices[1], pad, True)
    src_ext = jnp.concatenate(
        [src_flat, jnp.zeros((pt0 - pt1,), jnp.int32)])
    d0 = dst_flat[:l0].reshape(NS, kc0, 1, PCH)
    d1 = dst_flat[l0:].reshape(NS, kc1, 1, PCH)
    d1 = jnp.pad(d1, ((0, 0), (0, kc0 - kc1), (0, 0), (0, 0)),
                 constant_values=n)
    dst4p = jnp.concatenate([d0, d1], axis=0)

    k = -(-(l0 + l1) // (NW * LANES))
    if k % 2:
        k += 1
    pad_deg = NW * k * LANES - e
    dst4 = padded(edge_indices[1], pad_deg, True).reshape(NW, k, 1, LANES)

    zeros_acc = jnp.zeros((n_acc, f), jnp.float32)
    ones_rows = jnp.ones((LANES, f), jnp.float32)

    grid = -(-n // ROWBLK)

    degp = _sc_degree(n_acc, k, f)(dst4, ones_rows, zeros_acc)

    W1t = jnp.transpose(W1)
    W2t = jnp.transpose(W2)
    Wlt = jnp.transpose(Wl)

    y1, dinv = _tc_stage1(n, f, grid)(
        node_features, W1t, degp[0, :n], degp[1, :n])

    prop = _sc_propagate(n_acc, pt0, pt1, f)
    acc1 = prop(src_ext, dst4p, y1, zeros_acc)
    y2 = _tc_stage2(n, f, grid)(
        acc1[0, :n], acc1[1, :n], y1, dinv, b1.reshape(1, f), W2t)

    acc2 = prop(src_ext, dst4p, y2, zeros_acc)
    out = _tc_stage3(n, f, c_out, grid)(
        acc2[0, :n], acc2[1, :n], y2, dinv, b2.reshape(1, f), Wlt,
        bl.reshape(1, c_out))
    return out
